# baseline (speedup 1.0000x reference)
import functools

import jax
import jax.numpy as jnp
from jax import lax
from jax.experimental import pallas as pl
from jax.experimental.pallas import tpu as pltpu
from jax.experimental.pallas import tpu_sc as plsc

_NUM_REVIEWER = 100000
_NUM_DINER = 1000000
_EMB = 64
_BATCH = 16384
_NDIN = 5
_NC = 2
_NS = 16
_NW = _NC * _NS
_PER_W = _BATCH // _NW
_CHUNK = 128
_NCHUNK = _PER_W // _CHUNK
_L = 16


def _log1p_of(u):
    z = u / (u + 2.0)
    z2 = z * z
    p = 1.0 + z2 * (1.0 / 3.0 + z2 * (1.0 / 5.0 + z2 * (1.0 / 7.0 + z2 * (1.0 / 9.0))))
    return 2.0 * z * p


def _softplus(x):
    return jnp.maximum(x, 0.0) + _log1p_of(jnp.exp(-jnp.abs(x)))


def _body(rev_hbm, din_hbm, w_hbm, ridx_hbm, didx_hbm, out_hbm,
          ridx_v, didx_v, w_v, rev0, rev1, din0, din1, loss_v,
          sem0, sem1):
    wid = lax.axis_index("s") * _NC + lax.axis_index("c")
    iota = lax.iota(jnp.int32, _L)
    base = wid * _PER_W

    pltpu.sync_copy(ridx_hbm.at[pl.ds(base, _PER_W)], ridx_v)
    pltpu.sync_copy(didx_hbm.at[pl.ds(base * _NDIN, _PER_W * _NDIN)], didx_v)
    pltpu.sync_copy(w_hbm.at[pl.ds(base, _PER_W)], w_v)

    rev_bufs = (rev0, rev1)
    din_bufs = (din0, din1)
    sems = (sem0, sem1)

    def issue(c):
        b = c % 2
        return (
            pltpu.async_copy(
                rev_hbm.at[ridx_v.at[pl.ds(c * _CHUNK, _CHUNK)]],
                rev_bufs[b], sems[b]),
            pltpu.async_copy(
                din_hbm.at[didx_v.at[pl.ds(c * _CHUNK * _NDIN,
                                           _CHUNK * _NDIN)]],
                din_bufs[b], sems[b]),
        )

    inflight = issue(0)
    loss = jnp.zeros((_L,), jnp.float32)
    for c in range(_NCHUNK):
        nxt = issue(c + 1) if c + 1 < _NCHUNK else ()
        for cp in inflight:
            cp.wait()
        inflight = nxt
        rev_rows = rev_bufs[c % 2]
        din_rows = din_bufs[c % 2]

        def group_step(g, loss_acc, c=c, rev_rows=rev_rows, din_rows=din_rows):
            samp = g * _L + iota
            dbase = samp * _NDIN
            w = w_v[pl.ds(c * _CHUNK + g * _L, _L)]

            @plsc.parallel_loop(0, _EMB, 1, unroll=8,
                                carry=tuple(jnp.zeros((_L,), jnp.float32)
                                            for _ in range(_NDIN)))
            def accs(e, acc):
                col = jnp.broadcast_to(e, (_L,))
                rcol = plsc.load_gather(rev_rows, [samp, col])
                return tuple(
                    acc[d] + rcol * plsc.load_gather(din_rows, [dbase + d, col])
                    for d in range(_NDIN))
            contrib = jnp.zeros((_L,), jnp.float32)
            for d in range(1, _NDIN):
                contrib = contrib + _softplus(accs[d] - accs[0])
            return loss_acc + w * contrib

        loss = lax.fori_loop(0, _CHUNK // _L, group_step, loss)

    loss_v[...] = loss
    pltpu.sync_copy(loss_v, out_hbm.at[wid])


@jax.jit
def _run(rev_emb, din_emb, weights, reviewers, diners):
    mesh = plsc.VectorSubcoreMesh(core_axis_name="c", subcore_axis_name="s")
    partials = pl.kernel(
        _body,
        out_type=jax.ShapeDtypeStruct((_NW, _L), jnp.float32),
        mesh=mesh,
        scratch_types=[
            pltpu.VMEM((_PER_W,), jnp.int32),
            pltpu.VMEM((_PER_W * _NDIN,), jnp.int32),
            pltpu.VMEM((_PER_W,), jnp.float32),
            pltpu.VMEM((_CHUNK, _EMB), jnp.float32),
            pltpu.VMEM((_CHUNK, _EMB), jnp.float32),
            pltpu.VMEM((_CHUNK * _NDIN, _EMB), jnp.float32),
            pltpu.VMEM((_CHUNK * _NDIN, _EMB), jnp.float32),
            pltpu.VMEM((_L,), jnp.float32),
            pltpu.SemaphoreType.DMA,
            pltpu.SemaphoreType.DMA,
        ],
        compiler_params=pltpu.CompilerParams(
            needs_layout_passes=False, use_tc_tiling_on_sc=False),
    )(rev_emb, din_emb, weights, reviewers, diners)
    return jnp.sum(partials) * (1.0 / (_BATCH * (_NDIN - 1)))


def kernel(reviewer_emb, diner_emb, weights, reviewers, diners):
    return _run(
        reviewer_emb,
        diner_emb,
        jnp.reshape(weights, (_BATCH,)),
        reviewers.astype(jnp.int32),
        jnp.reshape(diners.astype(jnp.int32), (_BATCH * _NDIN,)),
    )


# speedup vs baseline: 1.0019x; 1.0019x over previous
import functools

import jax
import jax.numpy as jnp
from jax import lax
from jax.experimental import pallas as pl
from jax.experimental.pallas import tpu as pltpu
from jax.experimental.pallas import tpu_sc as plsc

_NUM_REVIEWER = 100000
_NUM_DINER = 1000000
_EMB = 64
_BATCH = 16384
_NDIN = 5
_NC = 2
_NS = 16
_NW = _NC * _NS
_PER_W = _BATCH // _NW
_CHUNK = 128
_NCHUNK = _PER_W // _CHUNK
_L = 16


def _log1p_of(u):
    z = u / (u + 2.0)
    z2 = z * z
    p = 1.0 + z2 * (1.0 / 3.0 + z2 * (1.0 / 5.0 + z2 * (1.0 / 7.0 + z2 * (1.0 / 9.0))))
    return 2.0 * z * p


def _softplus(x):
    return jnp.maximum(x, 0.0) + _log1p_of(jnp.exp(-jnp.abs(x)))


def _body(rev_hbm, din_hbm, w_hbm, ridx_hbm, didx_hbm, out_hbm,
          ridx_v, didx_v, w_v, rev0, rev1, din0, din1, loss_v,
          sem0, sem1):
    wid = lax.axis_index("s") * _NC + lax.axis_index("c")
    iota = lax.iota(jnp.int32, _L)
    base = wid * _PER_W

    pltpu.sync_copy(ridx_hbm.at[pl.ds(base, _PER_W)], ridx_v)
    pltpu.sync_copy(didx_hbm.at[pl.ds(base * _NDIN, _PER_W * _NDIN)], didx_v)
    pltpu.sync_copy(w_hbm.at[pl.ds(base, _PER_W)], w_v)

    rev_bufs = (rev0, rev1)
    din_bufs = (din0, din1)
    sems = (sem0, sem1)

    def issue(c):
        b = c % 2
        return (
            pltpu.async_copy(
                rev_hbm.at[ridx_v.at[pl.ds(c * _CHUNK, _CHUNK)]],
                rev_bufs[b], sems[b]),
            pltpu.async_copy(
                din_hbm.at[didx_v.at[pl.ds(c * _CHUNK * _NDIN,
                                           _CHUNK * _NDIN)]],
                din_bufs[b], sems[b]),
        )

    inflight = issue(0)
    loss = jnp.zeros((_L,), jnp.float32)
    for c in range(_NCHUNK):
        nxt = issue(c + 1) if c + 1 < _NCHUNK else ()
        for cp in inflight:
            cp.wait()
        inflight = nxt
        rev_rows = rev_bufs[c % 2]
        din_rows = din_bufs[c % 2]

        def group_step(g, loss_acc, c=c, rev_rows=rev_rows, din_rows=din_rows):
            samp = g * _L + iota
            dbase = samp * _NDIN
            w = w_v[pl.ds(c * _CHUNK + g * _L, _L)]

            def e_step(e4, accs):
                new = list(accs)
                for u in range(4):
                    col = jnp.broadcast_to(e4 * 4 + u, (_L,))
                    rcol = plsc.load_gather(rev_rows, [samp, col])
                    for d in range(_NDIN):
                        dcol = plsc.load_gather(din_rows, [dbase + d, col])
                        new[d] = new[d] + rcol * dcol
                return tuple(new)

            accs = lax.fori_loop(
                0, _EMB // 4, e_step,
                tuple(jnp.zeros((_L,), jnp.float32) for _ in range(_NDIN)))
            contrib = jnp.zeros((_L,), jnp.float32)
            for d in range(1, _NDIN):
                contrib = contrib + _softplus(accs[d] - accs[0])
            return loss_acc + w * contrib

        loss = lax.fori_loop(0, _CHUNK // _L, group_step, loss)

    loss_v[...] = loss
    pltpu.sync_copy(loss_v, out_hbm.at[wid])


@jax.jit
def _run(rev_emb, din_emb, weights, reviewers, diners):
    mesh = plsc.VectorSubcoreMesh(core_axis_name="c", subcore_axis_name="s")
    partials = pl.kernel(
        _body,
        out_type=jax.ShapeDtypeStruct((_NW, _L), jnp.float32),
        mesh=mesh,
        scratch_types=[
            pltpu.VMEM((_PER_W,), jnp.int32),
            pltpu.VMEM((_PER_W * _NDIN,), jnp.int32),
            pltpu.VMEM((_PER_W,), jnp.float32),
            pltpu.VMEM((_CHUNK, _EMB), jnp.float32),
            pltpu.VMEM((_CHUNK, _EMB), jnp.float32),
            pltpu.VMEM((_CHUNK * _NDIN, _EMB), jnp.float32),
            pltpu.VMEM((_CHUNK * _NDIN, _EMB), jnp.float32),
            pltpu.VMEM((_L,), jnp.float32),
            pltpu.SemaphoreType.DMA,
            pltpu.SemaphoreType.DMA,
        ],
        compiler_params=pltpu.CompilerParams(
            needs_layout_passes=False, use_tc_tiling_on_sc=False),
    )(rev_emb, din_emb, weights, reviewers, diners)
    return jnp.sum(partials) * (1.0 / (_BATCH * (_NDIN - 1)))


def kernel(reviewer_emb, diner_emb, weights, reviewers, diners):
    return _run(
        reviewer_emb,
        diner_emb,
        jnp.reshape(weights, (_BATCH,)),
        reviewers.astype(jnp.int32),
        jnp.reshape(diners.astype(jnp.int32), (_BATCH * _NDIN,)),
    )
